# baseline (speedup 1.0000x reference)
import jax
import jax.numpy as jnp
from jax.experimental import pallas as pl


TILE_N = 1024


def _chamfer_body(a_ref, b_ref, dist1_ref, dist2_ref):
    i = pl.program_id(1)
    d = jax.lax.dot_general(
        a_ref[0], b_ref[0], (((1,), (1,)), ((), ())),
        preferred_element_type=jnp.float32,
    )
    dist1_ref[0, :, :] = jnp.min(d, axis=1, keepdims=True)
    partial = jnp.min(d, axis=0, keepdims=True)

    @pl.when(i == 0)
    def _init():
        dist2_ref[0, :, :] = partial

    @pl.when(i > 0)
    def _acc():
        dist2_ref[0, :, :] = jnp.minimum(dist2_ref[0, :, :], partial)


def _split3_bf16(s):
    hi = s.astype(jnp.bfloat16).astype(jnp.float32)
    r1 = s - hi
    mid = r1.astype(jnp.bfloat16).astype(jnp.float32)
    lo = r1 - mid
    return hi, mid, lo


@jax.jit
def kernel(input1, input2):
    b, n, _ = input1.shape
    m = input2.shape[1]
    sq1 = jnp.sum(input1 * input1, axis=-1)
    sq2 = jnp.sum(input2 * input2, axis=-1)
    s1h, s1m, s1l = _split3_bf16(sq1)
    s2h, s2m, s2l = _split3_bf16(sq2)
    ones1 = jnp.ones((b, n, 3), jnp.float32)
    ones2 = jnp.ones((b, m, 3), jnp.float32)
    zeros1 = jnp.zeros((b, n, 7), jnp.float32)
    zeros2 = jnp.zeros((b, m, 7), jnp.float32)
    a = jnp.concatenate(
        [-2.0 * input1, s1h[..., None], s1m[..., None], s1l[..., None],
         ones1, zeros1], axis=-1).astype(jnp.bfloat16)
    bb = jnp.concatenate(
        [input2, ones2, s2h[..., None], s2m[..., None], s2l[..., None],
         zeros2], axis=-1).astype(jnp.bfloat16)

    grid = (b, n // TILE_N)
    dist1, dist2 = pl.pallas_call(
        _chamfer_body,
        grid=grid,
        in_specs=[
            pl.BlockSpec((1, TILE_N, 16), lambda bi, i: (bi, i, 0)),
            pl.BlockSpec((1, m, 16), lambda bi, i: (bi, 0, 0)),
        ],
        out_specs=[
            pl.BlockSpec((1, TILE_N, 1), lambda bi, i: (bi, i, 0)),
            pl.BlockSpec((1, 1, m), lambda bi, i: (bi, 0, 0)),
        ],
        out_shape=[
            jax.ShapeDtypeStruct((b, n, 1), jnp.float32),
            jax.ShapeDtypeStruct((b, 1, m), jnp.float32),
        ],
    )(a, bb)
    return dist1[:, :, 0], dist2[:, 0, :]
